# baseline (device time: 130365 ns/iter reference)
def kernel(x, router_W, route_idx, expert_W):
    import jax
    import jax.numpy as jnp
    from jax import lax
    from jax.experimental import pallas as pl
    from jax.experimental.pallas import tpu as pltpu

    N_DEV = 32
    N_EXP = 128
    EPG = 4
    CAP = 204
    K = 40
    M, D = x.shape
    H = expert_W.shape[-1]
    G = N_DEV * K
    S = EPG * G

    def body(x_ref, ridx_ref, ew_ref, out_ref,
             sbuf, rbuf, ybuf, rres, hbuf,
             hsend, dsend, ysend, hrecv, drecv, yrecv):
        my = lax.axis_index("i")

        barrier_sem = pltpu.get_barrier_semaphore()
        for r in range(N_DEV - 1):
            peer = lax.rem(my + 1 + r, N_DEV)
            pl.semaphore_signal(barrier_sem, inc=1, device_id=(peer,),
                                device_id_type=pl.DeviceIdType.MESH)
        pl.semaphore_wait(barrier_sem, N_DEV - 1)

        def mk(src, dst, ssem, rsem, dev):
            return pltpu.make_async_remote_copy(
                src_ref=src, dst_ref=dst, send_sem=ssem, recv_sem=rsem,
                device_id=(dev,), device_id_type=pl.DeviceIdType.MESH,
            )

        xbf = x_ref[...].astype(jnp.bfloat16)
        ridx = ridx_ref[...]

        e_iota = lax.broadcasted_iota(jnp.int32, (M, N_EXP), 1)
        oh = (ridx == e_iota).astype(jnp.float32)
        hist = jnp.sum(oh, axis=0).astype(jnp.int32)
        hbuf[N_DEV - 1, :] = hist

        h_rd = [mk(hbuf.at[N_DEV - 1], hbuf.at[N_DEV - 2 - r],
                   hsend.at[r], hrecv.at[0], lax.rem(my + 1 + r, N_DEV))
                for r in range(N_DEV - 1)]
        for rd in h_rd:
            rd.start()

        eq = ridx == ridx.reshape(1, M)
        i_row = lax.broadcasted_iota(jnp.int32, (M, M), 0)
        i_col = lax.broadcasted_iota(jnp.int32, (M, M), 1)
        rank = jnp.sum(
            jnp.logical_and(eq, i_col < i_row).astype(jnp.int32),
            axis=1, keepdims=True,
        )

        owner = lax.div(ridx, EPG)
        sub = lax.rem(ridx, EPG)
        rel = lax.rem(owner - my - 1 + 2 * N_DEV, N_DEV)
        slot = sub * G + rel * K + rank
        slot = jnp.where(rank < K, slot, S)
        slot_row = slot.reshape(1, M)

        d_rd = []
        for g in range(EPG):
            d_iota = lax.broadcasted_iota(jnp.int32, (G, M), 0) + g * G
            disp = (d_iota == slot_row).astype(jnp.bfloat16)
            sflat = jnp.dot(disp, xbf, preferred_element_type=jnp.float32)
            sbuf[g] = sflat.astype(jnp.bfloat16).reshape(N_DEV, K, D)
            rbuf[g, N_DEV - 1] = sbuf[g, N_DEV - 1]
            rds = [mk(sbuf.at[g, r], rbuf.at[g, N_DEV - 2 - r],
                      dsend.at[g, r], drecv.at[g],
                      lax.rem(my + 1 + r, N_DEV))
                   for r in range(N_DEV - 1)]
            for rd in rds:
                rd.start()
            d_rd.extend(rds)

        mk(hbuf.at[pl.ds(0, N_DEV - 1)], hbuf.at[pl.ds(0, N_DEV - 1)],
           hrecv.at[0], hrecv.at[0], my).wait_recv()
        r_iota = lax.broadcasted_iota(jnp.int32, (N_DEV - 1, N_EXP), 0)
        prior = jnp.where(r_iota >= N_DEV - 1 - my,
                          hbuf[pl.ds(0, N_DEV - 1), :], 0)
        off = jnp.sum(prior.astype(jnp.float32), axis=0)
        off_tok = jnp.sum(oh * off[None, :], axis=1, keepdims=True)
        accept = (off_tok + rank.astype(jnp.float32)) < CAP

        wbf = ew_ref[...].astype(jnp.bfloat16)
        y_rd = []
        for g in range(EPG):
            mk(rbuf.at[g, pl.ds(0, N_DEV - 1)],
               rbuf.at[g, pl.ds(0, N_DEV - 1)],
               drecv.at[g], drecv.at[g], my).wait_recv()
            toks = rbuf[g].reshape(N_DEV * K, D)
            y = jnp.dot(toks, wbf[g], preferred_element_type=jnp.float32)
            ybuf[g] = y.astype(jnp.bfloat16).reshape(N_DEV, K, H)
            rres[g, N_DEV - 1] = ybuf[g, N_DEV - 1]
            rds = [mk(ybuf.at[g, r], rres.at[g, N_DEV - 2 - r],
                      ysend.at[g, r], yrecv.at[g],
                      lax.rem(my + 1 + r, N_DEV))
                   for r in range(N_DEV - 1)]
            for rd in rds:
                rd.start()
            y_rd.extend(rds)

        res = jnp.zeros((M, H), jnp.float32)
        for g in range(EPG):
            mk(rres.at[g, pl.ds(0, N_DEV - 1)],
               rres.at[g, pl.ds(0, N_DEV - 1)],
               yrecv.at[g], yrecv.at[g], my).wait_recv()
            c_iota = lax.broadcasted_iota(jnp.int32, (M, G), 1) + g * G
            comb = (c_iota == slot).astype(jnp.bfloat16)
            res = res + jnp.dot(comb, rres[g].reshape(N_DEV * K, H),
                                preferred_element_type=jnp.float32)
        out_ref[...] = jnp.where(accept, res, 0.0)

        for rd in h_rd + d_rd + y_rd:
            rd.wait_send()

    out_shape = jax.ShapeDtypeStruct((M, H), jnp.float32)
    return pl.pallas_call(
        body,
        out_shape=out_shape,
        in_specs=[
            pl.BlockSpec(memory_space=pltpu.VMEM),
            pl.BlockSpec(memory_space=pltpu.VMEM),
            pl.BlockSpec(memory_space=pltpu.VMEM),
        ],
        out_specs=pl.BlockSpec(memory_space=pltpu.VMEM),
        scratch_shapes=[
            pltpu.VMEM((EPG, N_DEV, K, D), jnp.bfloat16),
            pltpu.VMEM((EPG, N_DEV, K, D), jnp.bfloat16),
            pltpu.VMEM((EPG, N_DEV, K, H), jnp.bfloat16),
            pltpu.VMEM((EPG, N_DEV, K, H), jnp.bfloat16),
            pltpu.VMEM((N_DEV, N_EXP), jnp.int32),
            pltpu.SemaphoreType.DMA((N_DEV - 1,)),
            pltpu.SemaphoreType.DMA((EPG, N_DEV - 1)),
            pltpu.SemaphoreType.DMA((EPG, N_DEV - 1)),
            pltpu.SemaphoreType.DMA((1,)),
            pltpu.SemaphoreType.DMA((EPG,)),
            pltpu.SemaphoreType.DMA((EPG,)),
        ],
        compiler_params=pltpu.CompilerParams(
            collective_id=0,
            vmem_limit_bytes=100 * 1024 * 1024,
        ),
    )(x, route_idx, expert_W)
